# Initial kernel scaffold; baseline (speedup 1.0000x reference)
import jax
import jax.numpy as jnp
from jax import lax
from jax.experimental import pallas as pl
from jax.experimental.pallas import tpu as pltpu
from jax.experimental.pallas import tpu_sc as plsc

NC = 2
NS = 16
NW = NC * NS
CHUNK = 128


def _prep_body(yt_ref, pt_ref, tt_ref, yi_ref, pi_ref, table_ref, idx_ref):
    table_ref[...] = jnp.concatenate(
        [yt_ref[...] + tt_ref[0:1, :], pt_ref[...] + tt_ref[1:2, :]], axis=0
    )
    V = yt_ref.shape[0]
    B, L = yi_ref.shape
    t = lax.broadcasted_iota(jnp.int32, (B, 2, L), 1)
    yi = jnp.broadcast_to(yi_ref[...][:, None, :], (B, 2, L))
    pi = jnp.broadcast_to(pi_ref[...][:, None, :] + V, (B, 2, L))
    idx_ref[...] = jnp.where(t == 0, yi, pi)


def _sc_gather_body(table_hbm, idx_hbm, out_hbm, idx_v, rows_v, sem):
    wid = lax.axis_index("s") * NC + lax.axis_index("c")
    n_chunks = idx_hbm.shape[0]
    cpw = n_chunks // NW
    c0 = wid * cpw

    def body(i, carry):
        c = c0 + i
        pltpu.sync_copy(idx_hbm.at[c], idx_v)
        pltpu.async_copy(table_hbm.at[idx_v], rows_v, sem).wait()
        pltpu.sync_copy(rows_v, out_hbm.at[pl.ds(c * CHUNK, CHUNK)])
        return carry

    lax.fori_loop(0, cpw, body, 0)


def kernel(yaw_ids, pitch_ids, yaw_table, pitch_table, type_table):
    B, L = yaw_ids.shape
    V, H = yaw_table.shape
    rows = B * 2 * L
    assert rows % (NW * CHUNK) == 0

    yaw_ids = yaw_ids.astype(jnp.int32)
    pitch_ids = pitch_ids.astype(jnp.int32)

    table, idx = pl.pallas_call(
        _prep_body,
        out_shape=(
            jax.ShapeDtypeStruct((2 * V, H), jnp.float32),
            jax.ShapeDtypeStruct((B, 2, L), jnp.int32),
        ),
    )(yaw_table, pitch_table, type_table, yaw_ids, pitch_ids)

    idx2 = idx.reshape(rows // CHUNK, CHUNK)

    mesh = plsc.VectorSubcoreMesh(
        core_axis_name="c", subcore_axis_name="s", num_cores=NC, num_subcores=NS
    )
    flat = pl.kernel(
        _sc_gather_body,
        out_type=jax.ShapeDtypeStruct((rows, H), jnp.float32),
        mesh=mesh,
        scratch_types=[
            pltpu.VMEM((CHUNK,), jnp.int32),
            pltpu.VMEM((CHUNK, H), jnp.float32),
            pltpu.SemaphoreType.DMA,
        ],
    )(table, idx2)

    return flat.reshape(B, 2, L, H)


# speedup vs baseline: 7.5645x; 7.5645x over previous
import jax
import jax.numpy as jnp
from jax import lax
from jax.experimental import pallas as pl
from jax.experimental.pallas import tpu as pltpu
from jax.experimental.pallas import tpu_sc as plsc

NC = 2
NS = 16
NW = NC * NS
CHUNK = 128


def _prep_body(yt_ref, pt_ref, tt_ref, yi_ref, pi_ref, table_ref, idx_ref):
    table_ref[...] = jnp.concatenate(
        [yt_ref[...] + tt_ref[0:1, :], pt_ref[...] + tt_ref[1:2, :]], axis=0
    )
    V = yt_ref.shape[0]
    B, L = yi_ref.shape
    t = lax.broadcasted_iota(jnp.int32, (B, 2, L), 1)
    yi = jnp.broadcast_to(yi_ref[...][:, None, :], (B, 2, L))
    pi = jnp.broadcast_to(pi_ref[...][:, None, :] + V, (B, 2, L))
    idx_ref[...] = jnp.where(t == 0, yi, pi)


def _sc_gather_body(table_hbm, idx_hbm, out_hbm, idx_v, rows_v, sem):
    wid = lax.axis_index("s") * NC + lax.axis_index("c")
    n_chunks = idx_hbm.shape[0]
    cpw = n_chunks // NW
    c0 = wid * cpw

    def body(i, carry):
        c = c0 + i
        pltpu.sync_copy(idx_hbm.at[c], idx_v)
        pltpu.async_copy(table_hbm.at[idx_v], rows_v, sem).wait()
        pltpu.sync_copy(rows_v, out_hbm.at[pl.ds(c * CHUNK, CHUNK)])
        return carry

    lax.fori_loop(0, cpw, body, 0)


def kernel(yaw_ids, pitch_ids, yaw_table, pitch_table, type_table):
    B, L = yaw_ids.shape
    V, H = yaw_table.shape
    rows = B * 2 * L
    assert rows % (NW * CHUNK) == 0

    yaw_ids = yaw_ids.astype(jnp.int32)
    pitch_ids = pitch_ids.astype(jnp.int32)

    table, idx = pl.pallas_call(
        _prep_body,
        out_shape=(
            jax.ShapeDtypeStruct((2 * V, H), jnp.float32),
            jax.ShapeDtypeStruct((B, 2, L), jnp.int32),
        ),
    )(yaw_table, pitch_table, type_table, yaw_ids, pitch_ids)

    idx2 = idx.reshape(rows // CHUNK, CHUNK)

    mesh = plsc.VectorSubcoreMesh(
        core_axis_name="c", subcore_axis_name="s", num_cores=NC, num_subcores=NS
    )
    flat = pl.kernel(
        _sc_gather_body,
        out_type=jax.ShapeDtypeStruct((rows, H), jnp.float32),
        mesh=mesh,
        scratch_types=[
            pltpu.VMEM((CHUNK,), jnp.int32),
            pltpu.VMEM((CHUNK, H), jnp.float32),
            pltpu.SemaphoreType.DMA,
        ],
        compiler_params=pltpu.CompilerParams(use_tc_tiling_on_sc=False),
    )(table, idx2)

    return flat.reshape(B, 2, L, H)


# speedup vs baseline: 10.9381x; 1.4460x over previous
import functools

import jax
import jax.numpy as jnp
from jax import lax
from jax.experimental import pallas as pl
from jax.experimental.pallas import tpu as pltpu
from jax.experimental.pallas import tpu_sc as plsc

NC = 2
NS = 16
LANES = 16

B = 4096
L = 50
V = 1024
H = 64

NQ = 4
QB = B // NQ
N_ITEMS = L * NQ
ITEMS_PER_TILE = -(-N_ITEMS // NS)


def _prep_body(yt_ref, pt_ref, tt_ref, tab_ref):
    ytT = jnp.swapaxes(yt_ref[...], 0, 1)
    ptT = jnp.swapaxes(pt_ref[...], 0, 1)
    ttT = jnp.swapaxes(tt_ref[...], 0, 1)
    a = ytT + ttT[:, 0:1]
    b = ptT + ttT[:, 1:2]
    tab_ref[pl.ds(0, V * H)] = a.reshape(V * H)
    tab_ref[pl.ds(V * H, V * H)] = b.reshape(V * H)


def _sc_body(tab_hbm, ids_hbm, out_hbm, tab_v, ids_v, obuf, sem):
    t = lax.axis_index("c")
    sid = lax.axis_index("s")

    pltpu.sync_copy(tab_hbm.at[pl.ds(t * (V * H), V * H)], tab_v)

    def item_body(k, carry):
        item = k * NS + sid

        @pl.when(item < N_ITEMS)
        def _():
            l = item // NQ
            q = item % NQ
            pltpu.sync_copy(ids_hbm.at[t * L + l, pl.ds(8 * q, 8)], ids_v)

            def bb_body(bb, c2):
                b0 = q * QB + bb * 128
                iv = [ids_v[bb, pl.ds(16 * j, 16)] for j in range(8)]
                for r in range(8):
                    for hh in range(8):
                        h = 8 * r + hh
                        for j in range(8):
                            vec = plsc.load_gather(tab_v, [iv[j] + h * V])
                            obuf[r, hh, pl.ds(16 * j, 16)] = vec
                copies = [
                    pltpu.async_copy(
                        obuf.at[r],
                        out_hbm.at[t, l, pl.ds(8 * r, 8), pl.ds(b0, 128)],
                        sem,
                    )
                    for r in range(8)
                ]
                for c in copies:
                    c.wait()
                return c2

            lax.fori_loop(0, 8, bb_body, 0)

        return carry

    lax.fori_loop(0, ITEMS_PER_TILE, item_body, 0)


def _impl(yaw_ids, pitch_ids, yaw_table, pitch_table, type_table):
    tab = pl.pallas_call(
        _prep_body,
        out_shape=jax.ShapeDtypeStruct((2 * V * H,), jnp.float32),
    )(yaw_table, pitch_table, type_table)

    ids = jnp.stack(
        [jnp.swapaxes(yaw_ids, 0, 1), jnp.swapaxes(pitch_ids, 0, 1)]
    ).astype(jnp.int32).reshape(2 * L, B // (32 * 128) * 32, 128)

    mesh = plsc.VectorSubcoreMesh(
        core_axis_name="c", subcore_axis_name="s", num_cores=NC, num_subcores=NS
    )
    out_t = pl.kernel(
        _sc_body,
        out_type=jax.ShapeDtypeStruct((2, L, H, B), jnp.float32),
        mesh=mesh,
        scratch_types=[
            pltpu.VMEM((V * H,), jnp.float32),
            pltpu.VMEM((8, 128), jnp.int32),
            pltpu.VMEM((8, 8, 128), jnp.float32),
            pltpu.SemaphoreType.DMA,
        ],
        compiler_params=pltpu.CompilerParams(
            use_tc_tiling_on_sc=True, needs_layout_passes=False
        ),
    )(tab, ids)

    return jnp.transpose(out_t, (3, 0, 1, 2))


def kernel(yaw_ids, pitch_ids, yaw_table, pitch_table, type_table):
    return _impl(yaw_ids, pitch_ids, yaw_table, pitch_table, type_table)


# speedup vs baseline: 11.6202x; 1.0624x over previous
import jax
import jax.numpy as jnp
from jax import lax
from jax.experimental import pallas as pl
from jax.experimental.pallas import tpu as pltpu
from jax.experimental.pallas import tpu_sc as plsc

NC = 2
NS = 16

B = 4096
L = 50
V = 1024
H = 64

NQ = 4
QB = B // NQ
N_ITEMS = L * NQ
ITEMS_PER_TILE = -(-N_ITEMS // NS)
PIECE_BYTES = 8 * 8 * 128 * 4


def _prep_body(yt_ref, pt_ref, tt_ref, tab_ref):
    ytT = jnp.swapaxes(yt_ref[...], 0, 1)
    ptT = jnp.swapaxes(pt_ref[...], 0, 1)
    ttT = jnp.swapaxes(tt_ref[...], 0, 1)
    a = ytT + ttT[:, 0:1]
    b = ptT + ttT[:, 1:2]
    tab_ref[pl.ds(0, V * H)] = a.reshape(V * H)
    tab_ref[pl.ds(V * H, V * H)] = b.reshape(V * H)


def _sc_body(tab_hbm, ids_hbm, out_hbm, tab_v, ids_v, obuf, sem):
    t = lax.axis_index("c")
    sid = lax.axis_index("s")

    pltpu.sync_copy(tab_hbm.at[pl.ds(t * (V * H), V * H)], tab_v)

    def drain_one():
        pltpu.make_async_copy(
            out_hbm.at[0, 0, 0, pl.ds(0, 8)], obuf.at[0], sem
        ).wait()

    def item_body(k, carry):
        item = k * NS + sid

        @pl.when(item < N_ITEMS)
        def _():
            l = item // NQ
            q = item % NQ
            pltpu.sync_copy(ids_hbm.at[t * L + l, pl.ds(8 * q, 8)], ids_v)

            def r_body(r, c2):
                slot = r % 2

                @pl.when(r >= 2)
                def _():
                    drain_one()

                hbase = r * 8 * V
                for cl in range(8):
                    iv = [ids_v[cl, pl.ds(16 * j, 16)] for j in range(8)]
                    for hh in range(8):
                        off = hbase + hh * V
                        for j in range(8):
                            vec = plsc.load_gather(tab_v, [iv[j] + off])
                            obuf[slot, cl, hh, pl.ds(16 * j, 16)] = vec
                pltpu.async_copy(
                    obuf.at[slot],
                    out_hbm.at[t, l, r, pl.ds(8 * q, 8)],
                    sem,
                )
                return c2

            lax.fori_loop(0, 8, r_body, 0)
            drain_one()
            drain_one()

        return carry

    lax.fori_loop(0, ITEMS_PER_TILE, item_body, 0)


def kernel(yaw_ids, pitch_ids, yaw_table, pitch_table, type_table):
    tab = pl.pallas_call(
        _prep_body,
        out_shape=jax.ShapeDtypeStruct((2 * V * H,), jnp.float32),
    )(yaw_table, pitch_table, type_table)

    ids = jnp.stack(
        [jnp.swapaxes(yaw_ids, 0, 1), jnp.swapaxes(pitch_ids, 0, 1)]
    ).astype(jnp.int32).reshape(2 * L, B // 128, 128)

    mesh = plsc.VectorSubcoreMesh(
        core_axis_name="c", subcore_axis_name="s", num_cores=NC, num_subcores=NS
    )
    out6 = pl.kernel(
        _sc_body,
        out_type=jax.ShapeDtypeStruct((2, L, H // 8, B // 128, 8, 128), jnp.float32),
        mesh=mesh,
        scratch_types=[
            pltpu.VMEM((V * H,), jnp.float32),
            pltpu.VMEM((8, 128), jnp.int32),
            pltpu.VMEM((2, 8, 8, 128), jnp.float32),
            pltpu.SemaphoreType.DMA,
        ],
        compiler_params=pltpu.CompilerParams(
            use_tc_tiling_on_sc=True, needs_layout_passes=False
        ),
    )(tab, ids)

    return jnp.transpose(out6, (3, 5, 0, 1, 2, 4)).reshape(B, 2, L, H)


# speedup vs baseline: 34.3677x; 2.9576x over previous
import jax
import jax.numpy as jnp
from jax import lax
from jax.experimental import pallas as pl
from jax.experimental.pallas import tpu as pltpu
from jax.experimental.pallas import tpu_sc as plsc

NC = 2
NS = 16

B = 4096
L = 50
V = 1024
H = 64

NQ = 4
QB = B // NQ
N_ITEMS = L * NQ
ITEMS_PER_TILE = -(-N_ITEMS // NS)
PIECE_BYTES = 8 * 8 * 128 * 4


def _prep_body(yt_ref, pt_ref, tt_ref, tab_ref):
    ytT = jnp.swapaxes(yt_ref[...], 0, 1)
    ptT = jnp.swapaxes(pt_ref[...], 0, 1)
    ttT = jnp.swapaxes(tt_ref[...], 0, 1)
    a = ytT + ttT[:, 0:1]
    b = ptT + ttT[:, 1:2]
    tab_ref[pl.ds(0, V * H)] = a.reshape(V * H)
    tab_ref[pl.ds(V * H, V * H)] = b.reshape(V * H)


def _sc_body(tab_hbm, ids_hbm, out_hbm, tab_v, ids_v, obuf, sem):
    t = lax.axis_index("c")
    sid = lax.axis_index("s")

    pltpu.sync_copy(tab_hbm.at[pl.ds(t * (V * H), V * H)], tab_v)

    def drain_one():
        pltpu.make_async_copy(
            out_hbm.at[0, 0, 0, pl.ds(0, 8)], obuf.at[0], sem
        ).wait()

    def item_body(k, carry):
        item = k * NS + sid

        @pl.when(item < N_ITEMS)
        def _():
            l = item // NQ
            q = item % NQ
            pltpu.sync_copy(ids_hbm.at[t * L + l, pl.ds(8 * q, 8)], ids_v)

            def r_body(r, c2):
                slot = r % 2

                @pl.when(r >= 2)
                def _():
                    drain_one()

                hbase = r * 8 * V
                for cl in range(8):
                    iv = [ids_v[cl, pl.ds(16 * j, 16)] for j in range(8)]
                    for hh in range(0, 8, 2):
                        vecs = [
                            plsc.load_gather(tab_v, [iv[j] + (hbase + h * V)])
                            for h in (hh, hh + 1)
                            for j in range(8)
                        ]
                        for jj, h in ((0, hh), (8, hh + 1)):
                            for j in range(8):
                                obuf[slot, cl, h, pl.ds(16 * j, 16)] = vecs[jj + j]
                pltpu.async_copy(
                    obuf.at[slot],
                    out_hbm.at[t, l, r, pl.ds(8 * q, 8)],
                    sem,
                )
                return c2

            lax.fori_loop(0, 8, r_body, 0)
            drain_one()
            drain_one()

        return carry

    lax.fori_loop(0, ITEMS_PER_TILE, item_body, 0)


def kernel(yaw_ids, pitch_ids, yaw_table, pitch_table, type_table):
    tab = pl.pallas_call(
        _prep_body,
        out_shape=jax.ShapeDtypeStruct((2 * V * H,), jnp.float32),
    )(yaw_table, pitch_table, type_table)

    ids = jnp.stack(
        [jnp.swapaxes(yaw_ids, 0, 1), jnp.swapaxes(pitch_ids, 0, 1)]
    ).astype(jnp.int32).reshape(2 * L, B // 128, 128)

    mesh = plsc.VectorSubcoreMesh(
        core_axis_name="c", subcore_axis_name="s", num_cores=NC, num_subcores=NS
    )
    out6 = pl.kernel(
        _sc_body,
        out_type=jax.ShapeDtypeStruct((2, L, H // 8, B // 128, 8, 128), jnp.float32),
        mesh=mesh,
        scratch_types=[
            pltpu.VMEM((V * H,), jnp.float32),
            pltpu.VMEM((8, 128), jnp.int32),
            pltpu.VMEM((2, 8, 8, 128), jnp.float32),
            pltpu.SemaphoreType.DMA,
        ],
        compiler_params=pltpu.CompilerParams(
            use_tc_tiling_on_sc=True, needs_layout_passes=False
        ),
    )(tab, ids)

    return jnp.transpose(out6, (3, 5, 0, 1, 2, 4)).reshape(B, 2, L, H)


# speedup vs baseline: 38.4964x; 1.1201x over previous
import jax
import jax.numpy as jnp
from jax import lax
from jax.experimental import pallas as pl
from jax.experimental.pallas import tpu as pltpu
from jax.experimental.pallas import tpu_sc as plsc

NC = 2
NS = 16

B = 4096
L = 50
V = 1024
H = 64

NQ = 4
N_ITEMS = L * NQ
ITEMS_PER_TILE = -(-N_ITEMS // NS)


def _prep_body(ytT_ref, ptT_ref, ttT_ref, yiT_ref, piT_ref, tab_ref, ids_ref):
    a = ytT_ref[...] + ttT_ref[:, 0:1]
    b = ptT_ref[...] + ttT_ref[:, 1:2]
    tab_ref[pl.ds(0, V * H)] = a.reshape(V * H)
    tab_ref[pl.ds(V * H, V * H)] = b.reshape(V * H)
    ids_ref[pl.ds(0, L)] = yiT_ref[...].reshape(L, B // 128, 128)
    ids_ref[pl.ds(L, L)] = piT_ref[...].reshape(L, B // 128, 128)


def _sc_body(tab_hbm, ids_hbm, out_hbm, tab_v, ids_v, obuf, sem, sem_i):
    t = lax.axis_index("c")
    sid = lax.axis_index("s")

    pltpu.sync_copy(tab_hbm.at[pl.ds(t * (V * H), V * H)], tab_v)

    def fire_ids(k_next):
        item_n = k_next * NS + sid

        @pl.when(item_n < N_ITEMS)
        def _():
            l_n = item_n // NQ
            q_n = item_n % NQ
            pltpu.async_copy(
                ids_hbm.at[t * L + l_n, pl.ds(8 * q_n, 8)],
                ids_v.at[k_next % 2],
                sem_i,
            )

    def drain_piece():
        pltpu.make_async_copy(
            out_hbm.at[0, 0, 0, pl.ds(0, 8)], obuf.at[0], sem
        ).wait()

    fire_ids(0)

    def item_body(k, carry):
        item = k * NS + sid

        @pl.when(item < N_ITEMS)
        def _():
            l = item // NQ
            q = item % NQ
            islot = k % 2
            fire_ids(k + 1)
            pltpu.make_async_copy(
                ids_hbm.at[0, pl.ds(0, 8)], ids_v.at[0], sem_i
            ).wait()

            def r_body(r, c2):
                slot = r % 3

                @pl.when(r >= 2)
                def _():
                    drain_piece()

                hbase = r * 8 * V
                for cl in range(8):
                    iv = [ids_v[islot, cl, pl.ds(16 * j, 16)] for j in range(8)]
                    for hh in range(0, 8, 2):
                        vecs = [
                            plsc.load_gather(tab_v, [iv[j] + (hbase + h * V)])
                            for h in (hh, hh + 1)
                            for j in range(8)
                        ]
                        for jj, h in ((0, hh), (8, hh + 1)):
                            for j in range(8):
                                obuf[slot, cl, h, pl.ds(16 * j, 16)] = vecs[jj + j]
                pltpu.async_copy(
                    obuf.at[slot],
                    out_hbm.at[t, l, r, pl.ds(8 * q, 8)],
                    sem,
                )
                return c2

            lax.fori_loop(0, 8, r_body, 0)
            drain_piece()
            drain_piece()

        return carry

    lax.fori_loop(0, ITEMS_PER_TILE, item_body, 0)


def kernel(yaw_ids, pitch_ids, yaw_table, pitch_table, type_table):
    tab, ids = pl.pallas_call(
        _prep_body,
        out_shape=(
            jax.ShapeDtypeStruct((2 * V * H,), jnp.float32),
            jax.ShapeDtypeStruct((2 * L, B // 128, 128), jnp.int32),
        ),
    )(
        jnp.swapaxes(yaw_table, 0, 1),
        jnp.swapaxes(pitch_table, 0, 1),
        jnp.swapaxes(type_table, 0, 1),
        jnp.swapaxes(yaw_ids, 0, 1).astype(jnp.int32),
        jnp.swapaxes(pitch_ids, 0, 1).astype(jnp.int32),
    )

    mesh = plsc.VectorSubcoreMesh(
        core_axis_name="c", subcore_axis_name="s", num_cores=NC, num_subcores=NS
    )
    out6 = pl.kernel(
        _sc_body,
        out_type=jax.ShapeDtypeStruct((2, L, H // 8, B // 128, 8, 128), jnp.float32),
        mesh=mesh,
        scratch_types=[
            pltpu.VMEM((V * H,), jnp.float32),
            pltpu.VMEM((2, 8, 128), jnp.int32),
            pltpu.VMEM((3, 8, 8, 128), jnp.float32),
            pltpu.SemaphoreType.DMA,
            pltpu.SemaphoreType.DMA,
        ],
        compiler_params=pltpu.CompilerParams(
            use_tc_tiling_on_sc=True, needs_layout_passes=False
        ),
    )(tab, ids)

    return jnp.transpose(out6, (3, 5, 0, 1, 2, 4)).reshape(B, 2, L, H)


# speedup vs baseline: 40.0877x; 1.0413x over previous
import jax
import jax.numpy as jnp
from jax import lax
from jax.experimental import pallas as pl
from jax.experimental.pallas import tpu as pltpu
from jax.experimental.pallas import tpu_sc as plsc

NC = 2
NS = 16

B = 4096
L = 50
V = 1024
H = 64

NQ = 4
N_ITEMS = L * NQ
ITEMS_PER_TILE = -(-N_ITEMS // NS)


def _prep_body(ytT_ref, ptT_ref, ttT_ref, yiT_ref, piT_ref, tab_ref, ids_ref):
    a = ytT_ref[...] + ttT_ref[:, 0:1]
    b = ptT_ref[...] + ttT_ref[:, 1:2]
    tab_ref[pl.ds(0, V * H)] = a.reshape(V * H)
    tab_ref[pl.ds(V * H, V * H)] = b.reshape(V * H)
    ids_ref[pl.ds(0, L)] = yiT_ref[...].reshape(L, B // 128, 128)
    ids_ref[pl.ds(L, L)] = piT_ref[...].reshape(L, B // 128, 128)


def _sc_body(tab_hbm, ids_hbm, out_hbm, tab_v, ids_v, obuf, sem, sem_i):
    t = lax.axis_index("c")
    sid = lax.axis_index("s")

    pltpu.sync_copy(tab_hbm.at[pl.ds(t * (V * H), V * H)], tab_v)

    def fire_ids(k_next):
        item_n = k_next * NS + sid

        @pl.when(item_n < N_ITEMS)
        def _():
            l_n = item_n // NQ
            q_n = item_n % NQ
            pltpu.async_copy(
                ids_hbm.at[t * L + l_n, pl.ds(8 * q_n, 8)],
                ids_v.at[k_next % 2],
                sem_i,
            )

    def drain_piece():
        pltpu.make_async_copy(
            out_hbm.at[0, 0, 0, pl.ds(0, 8)], obuf.at[0], sem
        ).wait()

    fire_ids(0)

    def item_body(k, carry):
        item = k * NS + sid

        @pl.when(item < N_ITEMS)
        def _():
            l = item // NQ
            q = item % NQ
            islot = k % 2
            fire_ids(k + 1)
            pltpu.make_async_copy(
                ids_hbm.at[0, pl.ds(0, 8)], ids_v.at[0], sem_i
            ).wait()

            def r_body(r, c2):
                p = k * 8 + r
                slot = p % 3

                @pl.when(p >= 2)
                def _():
                    drain_piece()

                hbase = r * 8 * V
                for cl in range(8):
                    iv = [ids_v[islot, cl, pl.ds(16 * j, 16)] for j in range(8)]
                    for hh in range(0, 8, 2):
                        vecs = [
                            plsc.load_gather(tab_v, [iv[j] + (hbase + h * V)])
                            for h in (hh, hh + 1)
                            for j in range(8)
                        ]
                        for jj, h in ((0, hh), (8, hh + 1)):
                            for j in range(8):
                                obuf[slot, cl, h, pl.ds(16 * j, 16)] = vecs[jj + j]
                pltpu.async_copy(
                    obuf.at[slot],
                    out_hbm.at[t, l, r, pl.ds(8 * q, 8)],
                    sem,
                )
                return c2

            lax.fori_loop(0, 8, r_body, 0)

        return carry

    lax.fori_loop(0, ITEMS_PER_TILE, item_body, 0)
    drain_piece()
    drain_piece()


def kernel(yaw_ids, pitch_ids, yaw_table, pitch_table, type_table):
    tab, ids = pl.pallas_call(
        _prep_body,
        out_shape=(
            jax.ShapeDtypeStruct((2 * V * H,), jnp.float32),
            jax.ShapeDtypeStruct((2 * L, B // 128, 128), jnp.int32),
        ),
    )(
        jnp.swapaxes(yaw_table, 0, 1),
        jnp.swapaxes(pitch_table, 0, 1),
        jnp.swapaxes(type_table, 0, 1),
        jnp.swapaxes(yaw_ids, 0, 1).astype(jnp.int32),
        jnp.swapaxes(pitch_ids, 0, 1).astype(jnp.int32),
    )

    mesh = plsc.VectorSubcoreMesh(
        core_axis_name="c", subcore_axis_name="s", num_cores=NC, num_subcores=NS
    )
    out6 = pl.kernel(
        _sc_body,
        out_type=jax.ShapeDtypeStruct((2, L, H // 8, B // 128, 8, 128), jnp.float32),
        mesh=mesh,
        scratch_types=[
            pltpu.VMEM((V * H,), jnp.float32),
            pltpu.VMEM((2, 8, 128), jnp.int32),
            pltpu.VMEM((3, 8, 8, 128), jnp.float32),
            pltpu.SemaphoreType.DMA,
            pltpu.SemaphoreType.DMA,
        ],
        compiler_params=pltpu.CompilerParams(
            use_tc_tiling_on_sc=True, needs_layout_passes=False
        ),
    )(tab, ids)

    return jnp.transpose(out6, (3, 5, 0, 1, 2, 4)).reshape(B, 2, L, H)


# speedup vs baseline: 40.1027x; 1.0004x over previous
import jax
import jax.numpy as jnp
from jax import lax
from jax.experimental import pallas as pl
from jax.experimental.pallas import tpu as pltpu
from jax.experimental.pallas import tpu_sc as plsc

NC = 2
NS = 16

B = 4096
L = 50
V = 1024
H = 64

NQ = 4
N_ITEMS = L * NQ
ITEMS_PER_TILE = -(-N_ITEMS // NS)


def _prep_body(ytT_ref, ptT_ref, ttT_ref, yiT_ref, piT_ref, tab_ref, ids_ref):
    a = ytT_ref[...] + ttT_ref[:, 0:1]
    b = ptT_ref[...] + ttT_ref[:, 1:2]
    tab_ref[pl.ds(0, V * H)] = a.reshape(V * H)
    tab_ref[pl.ds(V * H, V * H)] = b.reshape(V * H)
    ids_ref[pl.ds(0, L)] = yiT_ref[...].reshape(L, B // 128, 128)
    ids_ref[pl.ds(L, L)] = piT_ref[...].reshape(L, B // 128, 128)


def _sc_body(tab_hbm, ids_hbm, out_hbm, tab_v, ids_v, obuf, sem, sem_i):
    t = lax.axis_index("c")
    sid = lax.axis_index("s")

    pltpu.sync_copy(tab_hbm.at[pl.ds(t * (V * H), V * H)], tab_v)

    def fire_ids(k_next):
        item_n = k_next * NS + sid

        @pl.when(item_n < N_ITEMS)
        def _():
            l_n = item_n // NQ
            q_n = item_n % NQ
            pltpu.async_copy(
                ids_hbm.at[t * L + l_n, pl.ds(8 * q_n, 8)],
                ids_v.at[k_next % 2],
                sem_i,
            )

    def drain_piece():
        pltpu.make_async_copy(
            out_hbm.at[0, 0, 0, pl.ds(0, 8)], obuf.at[0], sem
        ).wait()

    fire_ids(0)

    def item_body(k, carry):
        item = k * NS + sid

        @pl.when(item < N_ITEMS)
        def _():
            l = item // NQ
            q = item % NQ
            islot = k % 2
            fire_ids(k + 1)
            pltpu.make_async_copy(
                ids_hbm.at[0, pl.ds(0, 8)], ids_v.at[0], sem_i
            ).wait()

            def r_body(r, c2):
                p = k * 8 + r
                slot = p % 6

                @pl.when(p >= 5)
                def _():
                    drain_piece()

                hbase = r * 8 * V
                for cl in range(8):
                    iv = [ids_v[islot, cl, pl.ds(16 * j, 16)] for j in range(8)]
                    for hh in range(0, 8, 2):
                        vecs = [
                            plsc.load_gather(tab_v, [iv[j] + (hbase + h * V)])
                            for h in (hh, hh + 1)
                            for j in range(8)
                        ]
                        for jj, h in ((0, hh), (8, hh + 1)):
                            for j in range(8):
                                obuf[slot, cl, h, pl.ds(16 * j, 16)] = vecs[jj + j]
                pltpu.async_copy(
                    obuf.at[slot],
                    out_hbm.at[t, l, r, pl.ds(8 * q, 8)],
                    sem,
                )
                return c2

            lax.fori_loop(0, 8, r_body, 0)

        return carry

    lax.fori_loop(0, ITEMS_PER_TILE, item_body, 0)
    for _ in range(5):
        drain_piece()


def kernel(yaw_ids, pitch_ids, yaw_table, pitch_table, type_table):
    tab, ids = pl.pallas_call(
        _prep_body,
        out_shape=(
            jax.ShapeDtypeStruct((2 * V * H,), jnp.float32),
            jax.ShapeDtypeStruct((2 * L, B // 128, 128), jnp.int32),
        ),
    )(
        jnp.swapaxes(yaw_table, 0, 1),
        jnp.swapaxes(pitch_table, 0, 1),
        jnp.swapaxes(type_table, 0, 1),
        jnp.swapaxes(yaw_ids, 0, 1).astype(jnp.int32),
        jnp.swapaxes(pitch_ids, 0, 1).astype(jnp.int32),
    )

    mesh = plsc.VectorSubcoreMesh(
        core_axis_name="c", subcore_axis_name="s", num_cores=NC, num_subcores=NS
    )
    out6 = pl.kernel(
        _sc_body,
        out_type=jax.ShapeDtypeStruct((2, L, H // 8, B // 128, 8, 128), jnp.float32),
        mesh=mesh,
        scratch_types=[
            pltpu.VMEM((V * H,), jnp.float32),
            pltpu.VMEM((2, 8, 128), jnp.int32),
            pltpu.VMEM((6, 8, 8, 128), jnp.float32),
            pltpu.SemaphoreType.DMA,
            pltpu.SemaphoreType.DMA,
        ],
        compiler_params=pltpu.CompilerParams(
            use_tc_tiling_on_sc=True, needs_layout_passes=False
        ),
    )(tab, ids)

    return jnp.transpose(out6, (3, 5, 0, 1, 2, 4)).reshape(B, 2, L, H)
